# baseline (speedup 1.0000x reference)
import functools

import jax
import jax.numpy as jnp
from jax.experimental import pallas as pl
from jax.experimental.pallas import tpu as pltpu

D_MODEL = 1024
N_HEADS = 16
N_KV_HEADS = 4
D_FF = 2816
NUM_EXPERTS = 8
TOP_K = 2
HEAD_DIM = D_MODEL // N_HEADS
MAX_SEQ_LEN = 2048

_HIGH = jax.lax.Precision.DEFAULT

TM = 128
XS_PAD = 4096 + NUM_EXPERTS * TM
BQ = 256


def _rope_apply(x, c2, s2, p64, n_heads):
    parts = []
    for h in range(n_heads):
        xh = x[:, h * HEAD_DIM:(h + 1) * HEAD_DIM]
        xr = jnp.dot(xh, p64, preferred_element_type=jnp.float32,
                     precision=_HIGH)
        parts.append(xh * c2 + xr * s2)
    return jnp.concatenate(parts, axis=-1)


def _qkv_body(x_ref, w_ref, wq_ref, wk_ref, wv_ref, c2_ref, s2_ref, p64_ref,
              q_ref, k_ref, v_ref):
    x = x_ref[...]
    xn = x * jax.lax.rsqrt(jnp.mean(x * x, axis=-1, keepdims=True) + 1e-6)
    xn = xn * w_ref[...]
    q = jnp.dot(xn, wq_ref[...], preferred_element_type=jnp.float32,
                precision=_HIGH)
    k = jnp.dot(xn, wk_ref[...], preferred_element_type=jnp.float32,
                precision=_HIGH)
    v = jnp.dot(xn, wv_ref[...], preferred_element_type=jnp.float32,
                precision=_HIGH)
    c2 = c2_ref[...]
    s2 = s2_ref[...]
    p64 = p64_ref[...]
    q_ref[...] = _rope_apply(q, c2, s2, p64, N_HEADS)
    k_ref[...] = _rope_apply(k, c2, s2, p64, N_KV_HEADS)
    v_ref[...] = v


def _qkv_call(x2d, w, Wq, Wk, Wv, C2, S2, P64):
    S = x2d.shape[0]
    KD = N_KV_HEADS * HEAD_DIM
    return pl.pallas_call(
        _qkv_body,
        grid=(S // BQ,),
        in_specs=[
            pl.BlockSpec((BQ, D_MODEL), lambda i: (i, 0)),
            pl.BlockSpec((1, D_MODEL), lambda i: (0, 0)),
            pl.BlockSpec((D_MODEL, D_MODEL), lambda i: (0, 0)),
            pl.BlockSpec((D_MODEL, KD), lambda i: (0, 0)),
            pl.BlockSpec((D_MODEL, KD), lambda i: (0, 0)),
            pl.BlockSpec((BQ, HEAD_DIM), lambda i: (i, 0)),
            pl.BlockSpec((BQ, HEAD_DIM), lambda i: (i, 0)),
            pl.BlockSpec((HEAD_DIM, HEAD_DIM), lambda i: (0, 0)),
        ],
        out_specs=(
            pl.BlockSpec((BQ, D_MODEL), lambda i: (i, 0)),
            pl.BlockSpec((BQ, KD), lambda i: (i, 0)),
            pl.BlockSpec((BQ, KD), lambda i: (i, 0)),
        ),
        out_shape=(
            jax.ShapeDtypeStruct((S, D_MODEL), jnp.float32),
            jax.ShapeDtypeStruct((S, KD), jnp.float32),
            jax.ShapeDtypeStruct((S, KD), jnp.float32),
        ),
    )(x2d, w, Wq, Wk, Wv, C2, S2, P64)


def _attn_body(q_ref, k_ref, v_ref, x_ref, wo_ref, o_ref):
    qb = pl.program_id(0)
    h = pl.program_id(1)
    q = q_ref[0]
    row = jax.lax.broadcasted_iota(jnp.int32, (BQ, BQ), 0)
    col = jax.lax.broadcasted_iota(jnp.int32, (BQ, BQ), 1)
    diag_mask = row >= col

    def kblock(kb, carry):
        m_run, l_run, acc = carry
        koff = pl.multiple_of(kb * BQ, BQ)
        k = k_ref[0, pl.ds(koff, BQ), :]
        v = v_ref[0, pl.ds(koff, BQ), :]
        s = jax.lax.dot_general(q, k, (((1,), (1,)), ((), ())),
                                preferred_element_type=jnp.float32,
                                precision=_HIGH) * (1.0 / 8.0)
        s = jnp.where(jnp.logical_or(kb < qb, diag_mask), s,
                      jnp.float32(-1e9))
        m_new = jnp.maximum(m_run, jnp.max(s, axis=-1, keepdims=True))
        p = jnp.exp(s - m_new)
        corr = jnp.exp(m_run - m_new)
        l_run = l_run * corr + jnp.sum(p, axis=-1, keepdims=True)
        acc = acc * corr + jnp.dot(p, v, preferred_element_type=jnp.float32,
                                   precision=_HIGH)
        return m_new, l_run, acc

    m0 = jnp.full((BQ, 1), -jnp.inf, jnp.float32)
    l0 = jnp.zeros((BQ, 1), jnp.float32)
    a0 = jnp.zeros((BQ, HEAD_DIM), jnp.float32)
    _, l_run, acc = jax.lax.fori_loop(0, qb + 1, kblock, (m0, l0, a0))
    o = acc / l_run
    contrib = jnp.dot(o, wo_ref[0], preferred_element_type=jnp.float32,
                      precision=_HIGH)

    @pl.when(h == 0)
    def _():
        o_ref[...] = x_ref[...] + contrib

    @pl.when(h > 0)
    def _():
        o_ref[...] = o_ref[...] + contrib


def _attn_call(q3, k3, v3, x2d, Wo3):
    S = x2d.shape[0]
    return pl.pallas_call(
        _attn_body,
        grid=(S // BQ, N_HEADS),
        in_specs=[
            pl.BlockSpec((1, BQ, HEAD_DIM), lambda qb, h: (h, qb, 0)),
            pl.BlockSpec((1, S, HEAD_DIM), lambda qb, h: (h // 4, 0, 0)),
            pl.BlockSpec((1, S, HEAD_DIM), lambda qb, h: (h // 4, 0, 0)),
            pl.BlockSpec((BQ, D_MODEL), lambda qb, h: (qb, 0)),
            pl.BlockSpec((1, HEAD_DIM, D_MODEL), lambda qb, h: (h, 0, 0)),
        ],
        out_specs=pl.BlockSpec((BQ, D_MODEL), lambda qb, h: (qb, 0)),
        out_shape=jax.ShapeDtypeStruct((S, D_MODEL), jnp.float32),
    )(q3, k3, v3, x2d, Wo3)


def _router_body(x_ref, w_ref, wr_ref, xn_ref, xnb_ref, pos_ref, wt_ref,
                 cnt_ref, off_ref, aux_ref):
    x1 = x_ref[...]
    xn = x1 * jax.lax.rsqrt(jnp.mean(x1 * x1, axis=-1, keepdims=True) + 1e-6)
    xn = xn * w_ref[...]
    xn_ref[...] = xn
    xnb_ref[...] = xn.astype(jnp.bfloat16)

    logits = jnp.dot(xn, wr_ref[...], preferred_element_type=jnp.float32,
                     precision=_HIGH)
    mx = jnp.max(logits, axis=-1, keepdims=True)
    ex = jnp.exp(logits - mx)
    probs = ex / jnp.sum(ex, axis=-1, keepdims=True)

    S = probs.shape[0]
    lane = jax.lax.broadcasted_iota(jnp.int32, (S, NUM_EXPERTS), 1)
    v1 = jnp.max(probs, axis=-1, keepdims=True)
    i1 = jnp.min(jnp.where(probs == v1, lane, NUM_EXPERTS), axis=-1,
                 keepdims=True)
    masked = jnp.where(lane == i1, jnp.float32(-1.0), probs)
    v2 = jnp.max(masked, axis=-1, keepdims=True)
    i2 = jnp.min(jnp.where(masked == v2, lane, NUM_EXPERTS), axis=-1,
                 keepdims=True)

    denom = v1 + v2
    wt_ref[...] = jnp.concatenate([v1 / denom, v2 / denom], axis=-1)

    oh1 = (lane == i1).astype(jnp.float32)
    oh2 = (lane == i2).astype(jnp.float32)
    c1 = jnp.sum(oh1, axis=0, keepdims=True)
    counts = c1 + jnp.sum(oh2, axis=0, keepdims=True)
    cnt_ref[...] = counts

    pmean = jnp.mean(probs, axis=0, keepdims=True)
    aux_ref[...] = (jnp.float32(NUM_EXPERTS) * jnp.sum(
        counts / jnp.float32(TOP_K * S) * pmean)).reshape(1, 1)

    G = S // 128
    E2 = 2 * NUM_EXPERTS
    oh12 = jnp.concatenate([oh1, oh2], axis=-1)
    ohr = oh12.reshape(G, 128, E2).astype(jnp.bfloat16)
    gr = jax.lax.broadcasted_iota(jnp.int32, (G, 128, 128), 1)
    gc = jax.lax.broadcasted_iota(jnp.int32, (G, 128, 128), 2)
    Lb = (gc < gr).astype(jnp.bfloat16)
    pre_local = jax.lax.dot_general(
        Lb, ohr, (((2,), (1,)), ((0,), (0,))),
        preferred_element_type=jnp.float32)
    gsum = jnp.sum(oh12.reshape(G, 128, E2), axis=1)
    rg = jax.lax.broadcasted_iota(jnp.int32, (G, G), 0)
    cg = jax.lax.broadcasted_iota(jnp.int32, (G, G), 1)
    Lg = (cg < rg).astype(jnp.float32)
    gpre = jnp.dot(Lg, gsum, preferred_element_type=jnp.float32)
    pre = (pre_local + gpre[:, None, :]).reshape(S, E2)
    pre1 = pre[:, :NUM_EXPERTS]
    pre2 = pre[:, NUM_EXPERTS:] + c1

    re_ = jax.lax.broadcasted_iota(jnp.int32, (NUM_EXPERTS, NUM_EXPERTS), 0)
    ce_ = jax.lax.broadcasted_iota(jnp.int32, (NUM_EXPERTS, NUM_EXPERTS), 1)
    U8 = (re_ < ce_).astype(jnp.float32)
    acnt = jnp.ceil(counts / TM) * TM
    offs = jnp.dot(acnt, U8, preferred_element_type=jnp.float32)
    off_ref[...] = offs.astype(jnp.int32)

    pos1 = jnp.sum(oh1 * (offs + pre1), axis=-1, keepdims=True)
    pos2 = jnp.sum(oh2 * (offs + pre2), axis=-1, keepdims=True)
    pos_ref[...] = jnp.concatenate([pos1, pos2], axis=-1).astype(jnp.int32)


def _router_call(x1, w, Wr):
    S = x1.shape[0]
    return pl.pallas_call(
        _router_body,
        out_shape=(
            jax.ShapeDtypeStruct((S, D_MODEL), jnp.float32),
            jax.ShapeDtypeStruct((S, D_MODEL), jnp.bfloat16),
            jax.ShapeDtypeStruct((S, TOP_K), jnp.int32),
            jax.ShapeDtypeStruct((S, TOP_K), jnp.float32),
            jax.ShapeDtypeStruct((1, NUM_EXPERTS), jnp.float32),
            jax.ShapeDtypeStruct((1, NUM_EXPERTS), jnp.int32),
            jax.ShapeDtypeStruct((1, 1), jnp.float32),
        ),
    )(x1, w, Wr)


def _ffn_body(off_ref, xs_ref, wg_ref, wu_ref, wd_ref, ws_ref, ys_ref):
    fb = pl.program_id(1)
    e = pl.program_id(0)
    start = off_ref[e]
    cnt = off_ref[NUM_EXPERTS + e]
    nt = (cnt + TM - 1) // TM

    def tile(t, _):
        s0 = pl.multiple_of(start + t * TM, TM)
        rows = xs_ref[pl.ds(s0, TM), :]
        g = jnp.dot(rows, wg_ref[0], preferred_element_type=jnp.float32)
        u = jnp.dot(rows, wu_ref[0], preferred_element_type=jnp.float32)
        h = (g * jax.lax.logistic(g) * u).astype(jnp.bfloat16)
        y = jnp.dot(h, wd_ref[0], preferred_element_type=jnp.float32)
        y = y * ws_ref[pl.ds(s0, TM), :]

        @pl.when(fb == 0)
        def _():
            ys_ref[pl.ds(s0, TM), :] = y

        @pl.when(fb == 1)
        def _():
            ys_ref[pl.ds(s0, TM), :] = ys_ref[pl.ds(s0, TM), :] + y

        return 0

    jax.lax.fori_loop(0, nt, tile, 0)


def _ffn_call(meta, xs, ws, Wgb, Wub, Wdb):
    HF = D_FF // 2
    grid_spec = pltpu.PrefetchScalarGridSpec(
        num_scalar_prefetch=1,
        grid=(NUM_EXPERTS, 2),
        in_specs=[
            pl.BlockSpec((XS_PAD, D_MODEL), lambda e, f, *_: (0, 0)),
            pl.BlockSpec((1, D_MODEL, HF), lambda e, f, *_: (e, 0, f)),
            pl.BlockSpec((1, D_MODEL, HF), lambda e, f, *_: (e, 0, f)),
            pl.BlockSpec((1, HF, D_MODEL), lambda e, f, *_: (e, f, 0)),
            pl.BlockSpec((XS_PAD, 1), lambda e, f, *_: (0, 0)),
        ],
        out_specs=pl.BlockSpec((XS_PAD, D_MODEL), lambda e, f, *_: (0, 0)),
    )
    return pl.pallas_call(
        _ffn_body,
        grid_spec=grid_spec,
        out_shape=jax.ShapeDtypeStruct((XS_PAD, D_MODEL), jnp.float32),
    )(meta, xs, Wgb, Wub, Wdb, ws)


def _rope_consts(S):
    inv_freq = 1.0 / (10000.0 ** (jnp.arange(0, HEAD_DIM, 2,
                                             dtype=jnp.float32) / HEAD_DIM))
    t = jnp.arange(MAX_SEQ_LEN, dtype=jnp.float32)
    freqs = jnp.outer(t, inv_freq)
    emb = jnp.concatenate([freqs, freqs], axis=-1)
    cos = jnp.cos(emb)[:S]
    sin = jnp.sin(emb)[:S]
    c = cos[:, 0::2]
    s = sin[:, 0::2]
    C2 = jnp.repeat(c, 2, axis=1)
    S2 = jnp.repeat(s, 2, axis=1)
    idx = jnp.arange(HEAD_DIM)
    P = jnp.zeros((HEAD_DIM, HEAD_DIM), jnp.float32)
    P = P.at[idx[1::2], idx[0::2]].set(-1.0)
    P = P.at[idx[0::2], idx[1::2]].set(1.0)
    return C2, S2, P


def kernel(x, attn_norm_w, ffn_norm_w, Wq, Wk, Wv, Wo, Wr, Wg, Wu, Wd):
    B, S, D = x.shape
    x2d = x.reshape(S, D)
    C2, S2, P64 = _rope_consts(S)

    q, k, v = _qkv_call(x2d, attn_norm_w.reshape(1, D), Wq, Wk, Wv,
                        C2, S2, P64)
    q3 = q.reshape(S, N_HEADS, HEAD_DIM).transpose(1, 0, 2)
    k3 = k.reshape(S, N_KV_HEADS, HEAD_DIM).transpose(1, 0, 2)
    v3 = v.reshape(S, N_KV_HEADS, HEAD_DIM).transpose(1, 0, 2)
    Wo3 = Wo.reshape(N_HEADS, HEAD_DIM, D_MODEL)

    x1 = _attn_call(q3, k3, v3, x2d, Wo3)

    xn2, xn2b, pos, wts, counts, offs, aux = _router_call(
        x1, ffn_norm_w.reshape(1, D), Wr)

    meta = jnp.concatenate([offs.reshape(NUM_EXPERTS),
                            counts.reshape(NUM_EXPERTS).astype(jnp.int32)])

    pos_flat = pos.reshape(TOP_K * S)
    tok = jnp.arange(TOP_K * S, dtype=jnp.int32) // TOP_K
    inv = jnp.zeros((XS_PAD,), jnp.int32).at[pos_flat].set(tok)
    xs = jnp.take(xn2b, inv, axis=0)
    ws = jnp.zeros((XS_PAD, 1), jnp.float32).at[pos_flat, 0].set(
        wts.reshape(TOP_K * S))

    Wgb = Wg.astype(jnp.bfloat16)
    Wub = Wu.astype(jnp.bfloat16)
    Wdb = Wd.astype(jnp.bfloat16)
    ys = _ffn_call(meta, xs, ws, Wgb, Wub, Wdb)

    out2d = x1 + ys[pos[:, 0]] + ys[pos[:, 1]]

    return (out2d.reshape(B, S, D), aux.reshape(()), counts.reshape(NUM_EXPERTS))


# speedup vs baseline: 1.1923x; 1.1923x over previous
import functools

import jax
import jax.numpy as jnp
from jax.experimental import pallas as pl
from jax.experimental.pallas import tpu as pltpu

D_MODEL = 1024
N_HEADS = 16
N_KV_HEADS = 4
D_FF = 2816
NUM_EXPERTS = 8
TOP_K = 2
HEAD_DIM = D_MODEL // N_HEADS
MAX_SEQ_LEN = 2048

_HIGH = jax.lax.Precision.DEFAULT

TM = 128
XS_PAD = 4096 + NUM_EXPERTS * TM
BQ = 256


def _rope_apply(x, c2, s2, p64, n_heads):
    parts = []
    for h in range(n_heads):
        xh = x[:, h * HEAD_DIM:(h + 1) * HEAD_DIM]
        xr = jnp.dot(xh, p64, preferred_element_type=jnp.float32,
                     precision=_HIGH)
        parts.append(xh * c2 + xr * s2)
    return jnp.concatenate(parts, axis=-1)


def _qkv_body(x_ref, w_ref, wq_ref, wk_ref, wv_ref, c2_ref, s2_ref, p64_ref,
              q_ref, k_ref, v_ref):
    x = x_ref[...]
    xn = x * jax.lax.rsqrt(jnp.mean(x * x, axis=-1, keepdims=True) + 1e-6)
    xn = xn * w_ref[...]
    q = jnp.dot(xn, wq_ref[...], preferred_element_type=jnp.float32,
                precision=_HIGH)
    k = jnp.dot(xn, wk_ref[...], preferred_element_type=jnp.float32,
                precision=_HIGH)
    v = jnp.dot(xn, wv_ref[...], preferred_element_type=jnp.float32,
                precision=_HIGH)
    c2 = c2_ref[...]
    s2 = s2_ref[...]
    p64 = p64_ref[...]
    q_ref[...] = _rope_apply(q, c2, s2, p64, N_HEADS)
    k_ref[...] = _rope_apply(k, c2, s2, p64, N_KV_HEADS)
    v_ref[...] = v


def _qkv_call(x2d, w, Wq, Wk, Wv, C2, S2, P64):
    S = x2d.shape[0]
    KD = N_KV_HEADS * HEAD_DIM
    return pl.pallas_call(
        _qkv_body,
        grid=(S // BQ,),
        in_specs=[
            pl.BlockSpec((BQ, D_MODEL), lambda i: (i, 0)),
            pl.BlockSpec((1, D_MODEL), lambda i: (0, 0)),
            pl.BlockSpec((D_MODEL, D_MODEL), lambda i: (0, 0)),
            pl.BlockSpec((D_MODEL, KD), lambda i: (0, 0)),
            pl.BlockSpec((D_MODEL, KD), lambda i: (0, 0)),
            pl.BlockSpec((BQ, HEAD_DIM), lambda i: (i, 0)),
            pl.BlockSpec((BQ, HEAD_DIM), lambda i: (i, 0)),
            pl.BlockSpec((HEAD_DIM, HEAD_DIM), lambda i: (0, 0)),
        ],
        out_specs=(
            pl.BlockSpec((BQ, D_MODEL), lambda i: (i, 0)),
            pl.BlockSpec((BQ, KD), lambda i: (i, 0)),
            pl.BlockSpec((BQ, KD), lambda i: (i, 0)),
        ),
        out_shape=(
            jax.ShapeDtypeStruct((S, D_MODEL), jnp.float32),
            jax.ShapeDtypeStruct((S, KD), jnp.float32),
            jax.ShapeDtypeStruct((S, KD), jnp.float32),
        ),
    )(x2d, w, Wq, Wk, Wv, C2, S2, P64)


def _attn_body(q_ref, k_ref, v_ref, x_ref, wo_ref, o_ref):
    qb = pl.program_id(0)
    h = pl.program_id(1)
    q = q_ref[0]
    NQB = k_ref.shape[1] // BQ

    for qbv in range(NQB):
        @pl.when(qb == qbv)
        def _(qbv=qbv):
            cols = (qbv + 1) * BQ
            k = k_ref[0, :cols, :]
            v = v_ref[0, :cols, :]
            s = jax.lax.dot_general(q, k, (((1,), (1,)), ((), ())),
                                    preferred_element_type=jnp.float32,
                                    precision=_HIGH) * (1.0 / 8.0)
            row = jax.lax.broadcasted_iota(jnp.int32, (BQ, cols), 0) \
                + qbv * BQ
            col = jax.lax.broadcasted_iota(jnp.int32, (BQ, cols), 1)
            s = jnp.where(row >= col, s, jnp.float32(-1e9))
            m = jnp.max(s, axis=-1, keepdims=True)
            p = jnp.exp(s - m)
            p = p / jnp.sum(p, axis=-1, keepdims=True)
            o = jnp.dot(p, v, preferred_element_type=jnp.float32,
                        precision=_HIGH)
            contrib = jnp.dot(o, wo_ref[0],
                              preferred_element_type=jnp.float32,
                              precision=_HIGH)

            @pl.when(h == 0)
            def _():
                o_ref[...] = x_ref[...] + contrib

            @pl.when(h > 0)
            def _():
                o_ref[...] = o_ref[...] + contrib


def _attn_call(q3, k3, v3, x2d, Wo3):
    S = x2d.shape[0]
    return pl.pallas_call(
        _attn_body,
        grid=(S // BQ, N_HEADS),
        in_specs=[
            pl.BlockSpec((1, BQ, HEAD_DIM), lambda qb, h: (h, qb, 0)),
            pl.BlockSpec((1, S, HEAD_DIM), lambda qb, h: (h // 4, 0, 0)),
            pl.BlockSpec((1, S, HEAD_DIM), lambda qb, h: (h // 4, 0, 0)),
            pl.BlockSpec((BQ, D_MODEL), lambda qb, h: (qb, 0)),
            pl.BlockSpec((1, HEAD_DIM, D_MODEL), lambda qb, h: (h, 0, 0)),
        ],
        out_specs=pl.BlockSpec((BQ, D_MODEL), lambda qb, h: (qb, 0)),
        out_shape=jax.ShapeDtypeStruct((S, D_MODEL), jnp.float32),
    )(q3, k3, v3, x2d, Wo3)


def _router_body(x_ref, w_ref, wr_ref, xn_ref, xnb_ref, pos_ref, wt_ref,
                 cnt_ref, off_ref, aux_ref):
    x1 = x_ref[...]
    xn = x1 * jax.lax.rsqrt(jnp.mean(x1 * x1, axis=-1, keepdims=True) + 1e-6)
    xn = xn * w_ref[...]
    xn_ref[...] = xn
    xnb_ref[...] = xn.astype(jnp.bfloat16)

    logits = jnp.dot(xn, wr_ref[...], preferred_element_type=jnp.float32,
                     precision=_HIGH)
    mx = jnp.max(logits, axis=-1, keepdims=True)
    ex = jnp.exp(logits - mx)
    probs = ex / jnp.sum(ex, axis=-1, keepdims=True)

    S = probs.shape[0]
    lane = jax.lax.broadcasted_iota(jnp.int32, (S, NUM_EXPERTS), 1)
    v1 = jnp.max(probs, axis=-1, keepdims=True)
    i1 = jnp.min(jnp.where(probs == v1, lane, NUM_EXPERTS), axis=-1,
                 keepdims=True)
    masked = jnp.where(lane == i1, jnp.float32(-1.0), probs)
    v2 = jnp.max(masked, axis=-1, keepdims=True)
    i2 = jnp.min(jnp.where(masked == v2, lane, NUM_EXPERTS), axis=-1,
                 keepdims=True)

    denom = v1 + v2
    wt_ref[...] = jnp.concatenate([v1 / denom, v2 / denom], axis=-1)

    oh1 = (lane == i1).astype(jnp.float32)
    oh2 = (lane == i2).astype(jnp.float32)
    c1 = jnp.sum(oh1, axis=0, keepdims=True)
    counts = c1 + jnp.sum(oh2, axis=0, keepdims=True)
    cnt_ref[...] = counts

    pmean = jnp.mean(probs, axis=0, keepdims=True)
    aux_ref[...] = (jnp.float32(NUM_EXPERTS) * jnp.sum(
        counts / jnp.float32(TOP_K * S) * pmean)).reshape(1, 1)

    G = S // 128
    E2 = 2 * NUM_EXPERTS
    oh12 = jnp.concatenate([oh1, oh2], axis=-1)
    ohr = oh12.reshape(G, 128, E2).astype(jnp.bfloat16)
    gr = jax.lax.broadcasted_iota(jnp.int32, (G, 128, 128), 1)
    gc = jax.lax.broadcasted_iota(jnp.int32, (G, 128, 128), 2)
    Lb = (gc < gr).astype(jnp.bfloat16)
    pre_local = jax.lax.dot_general(
        Lb, ohr, (((2,), (1,)), ((0,), (0,))),
        preferred_element_type=jnp.float32)
    gsum = jnp.sum(oh12.reshape(G, 128, E2), axis=1)
    rg = jax.lax.broadcasted_iota(jnp.int32, (G, G), 0)
    cg = jax.lax.broadcasted_iota(jnp.int32, (G, G), 1)
    Lg = (cg < rg).astype(jnp.float32)
    gpre = jnp.dot(Lg, gsum, preferred_element_type=jnp.float32)
    pre = (pre_local + gpre[:, None, :]).reshape(S, E2)
    pre1 = pre[:, :NUM_EXPERTS]
    pre2 = pre[:, NUM_EXPERTS:] + c1

    re_ = jax.lax.broadcasted_iota(jnp.int32, (NUM_EXPERTS, NUM_EXPERTS), 0)
    ce_ = jax.lax.broadcasted_iota(jnp.int32, (NUM_EXPERTS, NUM_EXPERTS), 1)
    U8 = (re_ < ce_).astype(jnp.float32)
    acnt = jnp.ceil(counts / TM) * TM
    offs = jnp.dot(acnt, U8, preferred_element_type=jnp.float32)
    off_ref[...] = offs.astype(jnp.int32)

    pos1 = jnp.sum(oh1 * (offs + pre1), axis=-1, keepdims=True)
    pos2 = jnp.sum(oh2 * (offs + pre2), axis=-1, keepdims=True)
    pos_ref[...] = jnp.concatenate([pos1, pos2], axis=-1).astype(jnp.int32)


def _router_call(x1, w, Wr):
    S = x1.shape[0]
    return pl.pallas_call(
        _router_body,
        out_shape=(
            jax.ShapeDtypeStruct((S, D_MODEL), jnp.float32),
            jax.ShapeDtypeStruct((S, D_MODEL), jnp.bfloat16),
            jax.ShapeDtypeStruct((S, TOP_K), jnp.int32),
            jax.ShapeDtypeStruct((S, TOP_K), jnp.float32),
            jax.ShapeDtypeStruct((1, NUM_EXPERTS), jnp.float32),
            jax.ShapeDtypeStruct((1, NUM_EXPERTS), jnp.int32),
            jax.ShapeDtypeStruct((1, 1), jnp.float32),
        ),
    )(x1, w, Wr)


def _ffn_body(off_ref, xs_ref, wg_ref, wu_ref, wd_ref, ws_ref, ys_ref):
    fb = pl.program_id(1)
    e = pl.program_id(0)
    start = off_ref[e]
    cnt = off_ref[NUM_EXPERTS + e]
    nt = (cnt + TM - 1) // TM

    def tile(t, _):
        s0 = pl.multiple_of(start + t * TM, TM)
        rows = xs_ref[pl.ds(s0, TM), :]
        g = jnp.dot(rows, wg_ref[0], preferred_element_type=jnp.float32)
        u = jnp.dot(rows, wu_ref[0], preferred_element_type=jnp.float32)
        h = (g * jax.lax.logistic(g) * u).astype(jnp.bfloat16)
        y = jnp.dot(h, wd_ref[0], preferred_element_type=jnp.float32)
        y = y * ws_ref[pl.ds(s0, TM), :]

        @pl.when(fb == 0)
        def _():
            ys_ref[pl.ds(s0, TM), :] = y

        @pl.when(fb == 1)
        def _():
            ys_ref[pl.ds(s0, TM), :] = ys_ref[pl.ds(s0, TM), :] + y

        return 0

    jax.lax.fori_loop(0, nt, tile, 0)


def _ffn_call(meta, xs, ws, Wgb, Wub, Wdb):
    HF = D_FF // 2
    grid_spec = pltpu.PrefetchScalarGridSpec(
        num_scalar_prefetch=1,
        grid=(NUM_EXPERTS, 2),
        in_specs=[
            pl.BlockSpec((XS_PAD, D_MODEL), lambda e, f, *_: (0, 0)),
            pl.BlockSpec((1, D_MODEL, HF), lambda e, f, *_: (e, 0, f)),
            pl.BlockSpec((1, D_MODEL, HF), lambda e, f, *_: (e, 0, f)),
            pl.BlockSpec((1, HF, D_MODEL), lambda e, f, *_: (e, f, 0)),
            pl.BlockSpec((XS_PAD, 1), lambda e, f, *_: (0, 0)),
        ],
        out_specs=pl.BlockSpec((XS_PAD, D_MODEL), lambda e, f, *_: (0, 0)),
    )
    return pl.pallas_call(
        _ffn_body,
        grid_spec=grid_spec,
        out_shape=jax.ShapeDtypeStruct((XS_PAD, D_MODEL), jnp.float32),
    )(meta, xs, Wgb, Wub, Wdb, ws)


def _rope_consts(S):
    inv_freq = 1.0 / (10000.0 ** (jnp.arange(0, HEAD_DIM, 2,
                                             dtype=jnp.float32) / HEAD_DIM))
    t = jnp.arange(MAX_SEQ_LEN, dtype=jnp.float32)
    freqs = jnp.outer(t, inv_freq)
    emb = jnp.concatenate([freqs, freqs], axis=-1)
    cos = jnp.cos(emb)[:S]
    sin = jnp.sin(emb)[:S]
    c = cos[:, 0::2]
    s = sin[:, 0::2]
    C2 = jnp.repeat(c, 2, axis=1)
    S2 = jnp.repeat(s, 2, axis=1)
    idx = jnp.arange(HEAD_DIM)
    P = jnp.zeros((HEAD_DIM, HEAD_DIM), jnp.float32)
    P = P.at[idx[1::2], idx[0::2]].set(-1.0)
    P = P.at[idx[0::2], idx[1::2]].set(1.0)
    return C2, S2, P


def kernel(x, attn_norm_w, ffn_norm_w, Wq, Wk, Wv, Wo, Wr, Wg, Wu, Wd):
    B, S, D = x.shape
    x2d = x.reshape(S, D)
    C2, S2, P64 = _rope_consts(S)

    q, k, v = _qkv_call(x2d, attn_norm_w.reshape(1, D), Wq, Wk, Wv,
                        C2, S2, P64)
    q3 = q.reshape(S, N_HEADS, HEAD_DIM).transpose(1, 0, 2)
    k3 = k.reshape(S, N_KV_HEADS, HEAD_DIM).transpose(1, 0, 2)
    v3 = v.reshape(S, N_KV_HEADS, HEAD_DIM).transpose(1, 0, 2)
    Wo3 = Wo.reshape(N_HEADS, HEAD_DIM, D_MODEL)

    x1 = _attn_call(q3, k3, v3, x2d, Wo3)

    xn2, xn2b, pos, wts, counts, offs, aux = _router_call(
        x1, ffn_norm_w.reshape(1, D), Wr)

    meta = jnp.concatenate([offs.reshape(NUM_EXPERTS),
                            counts.reshape(NUM_EXPERTS).astype(jnp.int32)])

    pos_flat = pos.reshape(TOP_K * S)
    tok = jnp.arange(TOP_K * S, dtype=jnp.int32) // TOP_K
    inv = jnp.zeros((XS_PAD,), jnp.int32).at[pos_flat].set(tok)
    xs = jnp.take(xn2b, inv, axis=0)
    ws = jnp.zeros((XS_PAD, 1), jnp.float32).at[pos_flat, 0].set(
        wts.reshape(TOP_K * S))

    Wgb = Wg.astype(jnp.bfloat16)
    Wub = Wu.astype(jnp.bfloat16)
    Wdb = Wd.astype(jnp.bfloat16)
    ys = _ffn_call(meta, xs, ws, Wgb, Wub, Wdb)

    out2d = x1 + ys[pos[:, 0]] + ys[pos[:, 1]]

    return (out2d.reshape(B, S, D), aux.reshape(()), counts.reshape(NUM_EXPERTS))
